# baseline (speedup 1.0000x reference)
import functools

import jax
import jax.numpy as jnp
from jax import lax
from jax.experimental import pallas as pl
from jax.experimental.pallas import tpu as pltpu
from jax.experimental.pallas import tpu_sc as plsc

VOCAB = 100000
HIST_LEN = 200
BATCH = 16384
PADDING_IDX = 0

NUM_CORES = 2
NUM_SUBCORES = 16
NW = NUM_CORES * NUM_SUBCORES
LANES = 16

BASE_COLS = HIST_LEN // NW
EXTRA = HIST_LEN - BASE_COLS * NW

OCHUNK = 4096
NOCHUNK = BATCH // OCHUNK
VREGS_PER_CHUNK = OCHUNK // LANES
UNROLL = 4

PBLK = 4096


@functools.cache
def _build_colgather():
    mesh = plsc.VectorSubcoreMesh(core_axis_name="c", subcore_axis_name="s")

    @functools.partial(
        pl.kernel,
        out_type=jax.ShapeDtypeStruct((HIST_LEN, BATCH), jnp.int32),
        mesh=mesh,
        compiler_params=pltpu.CompilerParams(
            use_tc_tiling_on_sc=True,
            needs_layout_passes=False,
        ),
        scratch_types=[
            pltpu.VMEM((BATCH,), jnp.int32),
            pltpu.VMEM((VOCAB,), jnp.int32),
            pltpu.VMEM((2, OCHUNK), jnp.int32),
            pltpu.SemaphoreType.DMA,
            pltpu.SemaphoreType.DMA,
            pltpu.SemaphoreType.DMA,
        ],
    )
    def _colgather(hist_t_hbm, anchor_hbm, out_hbm,
                   anch_v, col_v, out_v, csem, w0, w1):
        wid = lax.axis_index("s") * NUM_CORES + lax.axis_index("c")
        start = BASE_COLS * wid + jnp.minimum(wid, EXTRA)
        count = BASE_COLS + (wid < EXTRA).astype(jnp.int32)
        pltpu.sync_copy(anchor_hbm, anch_v)
        wsem = (w0, w1)

        def col_body(i, _):
            c = start + i
            pltpu.async_copy(hist_t_hbm.at[c], col_v, csem).wait()
            whandles = [None] * NOCHUNK
            for k in range(NOCHUNK):
                if k >= 2:
                    whandles[k - 2].wait()

                def vec_body(j, _):
                    b = k * OCHUNK + j * (LANES * UNROLL)
                    for u in range(UNROLL):
                        off = b + u * LANES
                        idx16 = anch_v[pl.ds(off, LANES)]
                        out_v[k & 1, pl.ds(j * (LANES * UNROLL) + u * LANES,
                                           LANES)] = (
                            plsc.load_gather(col_v, [idx16]))
                    return 0

                lax.fori_loop(0, VREGS_PER_CHUNK // UNROLL, vec_body, 0)
                whandles[k] = pltpu.async_copy(
                    out_v.at[k & 1],
                    out_hbm.at[c, pl.ds(k * OCHUNK, OCHUNK)],
                    wsem[k & 1])
            whandles[NOCHUNK - 2].wait()
            whandles[NOCHUNK - 1].wait()
            return 0

        lax.fori_loop(0, count, col_body, 0)

    return _colgather


def _mask_body(g_ref, t_ref, p_ref, m_ref):
    h = g_ref[...]
    tt = t_ref[...]
    keep = (h != tt) & (h != PADDING_IDX)
    p_ref[...] = jnp.where(keep, h, PADDING_IDX)
    m_ref[...] = keep


@functools.cache
def _build_mask():
    grid = BATCH // PBLK
    return pl.pallas_call(
        _mask_body,
        grid=(grid,),
        in_specs=[
            pl.BlockSpec((HIST_LEN, PBLK), lambda i: (0, i)),
            pl.BlockSpec((1, PBLK), lambda i: (0, i)),
        ],
        out_specs=[
            pl.BlockSpec((HIST_LEN, PBLK), lambda i: (0, i)),
            pl.BlockSpec((HIST_LEN, PBLK), lambda i: (0, i)),
        ],
        out_shape=(
            jax.ShapeDtypeStruct((HIST_LEN, BATCH), jnp.int32),
            jax.ShapeDtypeStruct((HIST_LEN, BATCH), jnp.bool_),
        ),
    )


def kernel(histories, anchor_idx, target_idx):
    out_dtype = histories.dtype
    hist_t = histories.astype(jnp.int32).T
    gathered_t = _build_colgather()(hist_t, anchor_idx.astype(jnp.int32))
    tgt_row = target_idx.astype(jnp.int32).reshape(1, BATCH)
    padded_t, mask_t = _build_mask()(gathered_t, tgt_row)
    return padded_t.T.astype(out_dtype), mask_t.T


# speedup vs baseline: 1.3001x; 1.3001x over previous
import functools

import jax
import jax.numpy as jnp
from jax import lax
from jax.experimental import pallas as pl
from jax.experimental.pallas import tpu as pltpu
from jax.experimental.pallas import tpu_sc as plsc

VOCAB = 100000
HIST_LEN = 200
PAD_LEN = 256
BATCH = 16384
PADDING_IDX = 0

NUM_CORES = 2
NUM_SUBCORES = 16
NW = NUM_CORES * NUM_SUBCORES
ROWS_PER_W = BATCH // NW
CHUNK = 128
NCHUNK = ROWS_PER_W // CHUNK

TBLK = 16384
PBLK = 8192


def _transpose_body(in_ref, out_ref):
    x = in_ref[...]
    out_ref[:, :HIST_LEN] = x.T


@functools.cache
def _build_transpose():
    grid = (VOCAB + TBLK - 1) // TBLK
    return pl.pallas_call(
        _transpose_body,
        grid=(grid,),
        in_specs=[pl.BlockSpec((HIST_LEN, TBLK), lambda i: (0, i))],
        out_specs=pl.BlockSpec((TBLK, PAD_LEN), lambda i: (i, 0)),
        out_shape=jax.ShapeDtypeStruct((VOCAB, PAD_LEN), jnp.int32),
    )


@functools.cache
def _build_gather():
    mesh = plsc.VectorSubcoreMesh(core_axis_name="c", subcore_axis_name="s")

    @functools.partial(
        pl.kernel,
        out_type=jax.ShapeDtypeStruct((BATCH, PAD_LEN), jnp.int32),
        mesh=mesh,
        compiler_params=pltpu.CompilerParams(
            use_tc_tiling_on_sc=True,
            needs_layout_passes=False,
        ),
        scratch_types=[
            pltpu.VMEM((NCHUNK, CHUNK), jnp.int32),
            pltpu.VMEM((CHUNK, PAD_LEN), jnp.int32),
            pltpu.VMEM((CHUNK, PAD_LEN), jnp.int32),
            pltpu.SemaphoreType.DMA,
            pltpu.SemaphoreType.DMA,
            pltpu.SemaphoreType.DMA,
            pltpu.SemaphoreType.DMA,
        ],
    )
    def _gather(hist_hbm, anchor_hbm, out_hbm,
                idx_v, rows0, rows1, g0, g1, w0, w1):
        wid = lax.axis_index("s") * NUM_CORES + lax.axis_index("c")
        base = wid * ROWS_PER_W
        for c in range(NCHUNK):
            pltpu.sync_copy(anchor_hbm.at[pl.ds(base + c * CHUNK, CHUNK)],
                            idx_v.at[c])
        rows = (rows0, rows1)
        gsem = (g0, g1)
        wsem = (w0, w1)
        ghandles = [None] * NCHUNK
        whandles = [None] * NCHUNK
        ghandles[0] = pltpu.async_copy(hist_hbm.at[idx_v.at[0]], rows[0],
                                       gsem[0])
        for c in range(NCHUNK):
            cur = c & 1
            ghandles[c].wait()
            whandles[c] = pltpu.async_copy(
                rows[cur], out_hbm.at[pl.ds(base + c * CHUNK, CHUNK)],
                wsem[cur])
            if c + 1 < NCHUNK:
                if c >= 1:
                    whandles[c - 1].wait()
                ghandles[c + 1] = pltpu.async_copy(
                    hist_hbm.at[idx_v.at[c + 1]], rows[1 - cur],
                    gsem[1 - cur])
        whandles[NCHUNK - 2].wait()
        whandles[NCHUNK - 1].wait()

    return _gather


def _mask_body(g_ref, t_ref, p_ref, m_ref):
    ht = g_ref[:, :HIST_LEN].T
    tt = t_ref[...]
    keep = (ht != tt) & (ht != PADDING_IDX)
    p_ref[...] = jnp.where(keep, ht, PADDING_IDX)
    m_ref[...] = keep


@functools.cache
def _build_mask():
    grid = BATCH // PBLK
    return pl.pallas_call(
        _mask_body,
        grid=(grid,),
        in_specs=[
            pl.BlockSpec((PBLK, PAD_LEN), lambda i: (i, 0)),
            pl.BlockSpec((1, PBLK), lambda i: (0, i)),
        ],
        out_specs=[
            pl.BlockSpec((HIST_LEN, PBLK), lambda i: (0, i)),
            pl.BlockSpec((HIST_LEN, PBLK), lambda i: (0, i)),
        ],
        out_shape=(
            jax.ShapeDtypeStruct((HIST_LEN, BATCH), jnp.int32),
            jax.ShapeDtypeStruct((HIST_LEN, BATCH), jnp.bool_),
        ),
    )


def kernel(histories, anchor_idx, target_idx):
    out_dtype = histories.dtype
    hist_t = histories.astype(jnp.int32).T
    hist_pad = _build_transpose()(hist_t)
    gathered = _build_gather()(hist_pad, anchor_idx.astype(jnp.int32))
    tgt_row = target_idx.astype(jnp.int32).reshape(1, BATCH)
    padded_t, mask_t = _build_mask()(gathered, tgt_row)
    return padded_t.T.astype(out_dtype), mask_t.T


# speedup vs baseline: 1.3176x; 1.0135x over previous
import functools

import jax
import jax.numpy as jnp
from jax import lax
from jax.experimental import pallas as pl
from jax.experimental.pallas import tpu as pltpu
from jax.experimental.pallas import tpu_sc as plsc

VOCAB = 100000
HIST_LEN = 200
PAD_LEN = 256
BATCH = 16384
PADDING_IDX = 0

NUM_CORES = 2
NUM_SUBCORES = 16
NW = NUM_CORES * NUM_SUBCORES
ROWS_PER_W = BATCH // NW
CHUNK = 128
NCHUNK = ROWS_PER_W // CHUNK

TBLK = 16384
PBLK = 8192


def _transpose_body(in_ref, out_ref):
    x = in_ref[...]
    out_ref[:, :HIST_LEN] = x.T


@functools.cache
def _build_transpose():
    grid = (VOCAB + TBLK - 1) // TBLK
    return pl.pallas_call(
        _transpose_body,
        grid=(grid,),
        in_specs=[pl.BlockSpec((HIST_LEN, TBLK), lambda i: (0, i))],
        out_specs=pl.BlockSpec((TBLK, PAD_LEN), lambda i: (i, 0)),
        out_shape=jax.ShapeDtypeStruct((VOCAB, PAD_LEN), jnp.int32),
    )


@functools.cache
def _build_gather():
    mesh = plsc.VectorSubcoreMesh(core_axis_name="c", subcore_axis_name="s")

    @functools.partial(
        pl.kernel,
        out_type=jax.ShapeDtypeStruct((BATCH, PAD_LEN), jnp.int32),
        mesh=mesh,
        compiler_params=pltpu.CompilerParams(
            use_tc_tiling_on_sc=True,
            needs_layout_passes=False,
        ),
        scratch_types=[
            pltpu.VMEM((NCHUNK, CHUNK), jnp.int32),
            pltpu.VMEM((CHUNK, PAD_LEN), jnp.int32),
            pltpu.VMEM((CHUNK, PAD_LEN), jnp.int32),
            pltpu.SemaphoreType.DMA,
            pltpu.SemaphoreType.DMA,
            pltpu.SemaphoreType.DMA,
            pltpu.SemaphoreType.DMA,
        ],
    )
    def _gather(hist_hbm, anchor_hbm, out_hbm,
                idx_v, rows0, rows1, g0, g1, w0, w1):
        wid = lax.axis_index("s") * NUM_CORES + lax.axis_index("c")
        base = wid * ROWS_PER_W
        for c in range(NCHUNK):
            pltpu.sync_copy(anchor_hbm.at[pl.ds(base + c * CHUNK, CHUNK)],
                            idx_v.at[c])
        rows = (rows0, rows1)
        gsem = (g0, g1)
        wsem = (w0, w1)
        ghandles = [None] * NCHUNK
        whandles = [None] * NCHUNK
        ghandles[0] = pltpu.async_copy(hist_hbm.at[idx_v.at[0]], rows[0],
                                       gsem[0])
        ghandles[1] = pltpu.async_copy(hist_hbm.at[idx_v.at[1]], rows[1],
                                       gsem[1])
        for c in range(NCHUNK):
            cur = c & 1
            ghandles[c].wait()
            whandles[c] = pltpu.async_copy(
                rows[cur], out_hbm.at[pl.ds(base + c * CHUNK, CHUNK)],
                wsem[cur])
            if c + 2 < NCHUNK:
                whandles[c].wait()
                ghandles[c + 2] = pltpu.async_copy(
                    hist_hbm.at[idx_v.at[c + 2]], rows[cur],
                    gsem[cur])
        whandles[NCHUNK - 2].wait()
        whandles[NCHUNK - 1].wait()

    return _gather


def _mask_body(g_ref, t_ref, p_ref, m_ref):
    ht = g_ref[:, :HIST_LEN].T
    tt = t_ref[...]
    keep = (ht != tt) & (ht != PADDING_IDX)
    p_ref[...] = jnp.where(keep, ht, PADDING_IDX)
    m_ref[...] = keep


@functools.cache
def _build_mask():
    grid = BATCH // PBLK
    return pl.pallas_call(
        _mask_body,
        grid=(grid,),
        in_specs=[
            pl.BlockSpec((PBLK, PAD_LEN), lambda i: (i, 0)),
            pl.BlockSpec((1, PBLK), lambda i: (0, i)),
        ],
        out_specs=[
            pl.BlockSpec((HIST_LEN, PBLK), lambda i: (0, i)),
            pl.BlockSpec((HIST_LEN, PBLK), lambda i: (0, i)),
        ],
        out_shape=(
            jax.ShapeDtypeStruct((HIST_LEN, BATCH), jnp.int32),
            jax.ShapeDtypeStruct((HIST_LEN, BATCH), jnp.bool_),
        ),
    )


def kernel(histories, anchor_idx, target_idx):
    out_dtype = histories.dtype
    hist_t = histories.astype(jnp.int32).T
    hist_pad = _build_transpose()(hist_t)
    gathered = _build_gather()(hist_pad, anchor_idx.astype(jnp.int32))
    tgt_row = target_idx.astype(jnp.int32).reshape(1, BATCH)
    padded_t, mask_t = _build_mask()(gathered, tgt_row)
    return padded_t.T.astype(out_dtype), mask_t.T
